# Initial kernel scaffold; baseline (speedup 1.0000x reference)
import jax
import jax.numpy as jnp
from jax.experimental import pallas as pl


def kernel(up, usol, ut, x_to_train_f, ut1, n_index, inv_mat):
    raise NotImplementedError("write your pallas kernel here")


# speedup vs baseline: 72.8525x; 72.8525x over previous
import jax
import jax.numpy as jnp
from jax import lax
from jax.experimental import pallas as pl
from jax.experimental.pallas import tpu as pltpu, tpu_sc as plsc

N = 100000
M = 9
NC = 2
NS = 16
NW = NC * NS
BLK = 160
NBLK = 20
IDXB = BLK * M
CPT = BLK * NBLK
NP = NW * CPT

_params = pltpu.CompilerParams(
    needs_layout_passes=False, use_tc_tiling_on_sc=False
)


def _make_mesh():
    return plsc.VectorSubcoreMesh(
        core_axis_name="c", subcore_axis_name="s", num_cores=NC, num_subcores=NS
    )


def _wid():
    return lax.axis_index("s") * NC + lax.axis_index("c")


def _full16(v):
    return jnp.full((16,), v, jnp.int32)


def _phase1(table1, nidx, own, out_t2, out_gp,
            idx_a, idx_b, buf_a, buf_b, own_a, own_b, outb, outp,
            sga, sgb, soa, sob):
    w = _wid()
    iota = lax.iota(jnp.int32, 16)
    sets = [(idx_a, buf_a, own_a, sga, soa), (idx_b, buf_b, own_b, sgb, sob)]

    def issue(b, s):
        idx_v, buf, ownv, sg, so = s
        pltpu.sync_copy(nidx.at[pl.ds((w * NBLK + b) * IDXB, IDXB)], idx_v)
        pltpu.async_copy(table1.at[idx_v], buf, sg)
        base = w * CPT + b * BLK
        pltpu.async_copy(own.at[:8, pl.ds(base, BLK)], ownv, so)

    def wait(s):
        idx_v, buf, ownv, sg, so = s
        pltpu.make_async_copy(table1.at[idx_v], buf, sg).wait()
        pltpu.make_async_copy(own.at[:8, pl.ds(0, BLK)], ownv, so).wait()

    def compute(b, s):
        idx_v, buf, ownv, sg, so = s
        base = w * CPT + b * BLK
        for p in range(0, BLK, 16):
            rows = iota + p
            xi0 = ownv[0, pl.ds(p, 16)]
            xi1 = ownv[1, pl.ds(p, 16)]
            uti = ownv[2, pl.ds(p, 16)]
            u1i = ownv[3, pl.ds(p, 16)]
            s00 = jnp.zeros((16,), jnp.float32)
            s01 = jnp.zeros((16,), jnp.float32)
            s10 = jnp.zeros((16,), jnp.float32)
            s11 = jnp.zeros((16,), jnp.float32)
            for m in range(M):
                rm = rows + m * BLK
                xj0 = plsc.load_gather(buf, [rm, _full16(0)])
                xj1 = plsc.load_gather(buf, [rm, _full16(1)])
                uj = plsc.load_gather(buf, [rm, _full16(2)])
                u1j = plsc.load_gather(buf, [rm, _full16(3)])
                xd0 = xj0 - xi0
                xd1 = xj1 - xi1
                ud = uj - uti
                ud1 = u1j - u1i
                s00 = s00 + ud * xd0
                s01 = s01 + ud * xd1
                s10 = s10 + ud1 * xd0
                s11 = s11 + ud1 * xd1
            ia = ownv[4, pl.ds(p, 16)]
            ib = ownv[5, pl.ds(p, 16)]
            ic = ownv[6, pl.ds(p, 16)]
            id_ = ownv[7, pl.ds(p, 16)]
            gx = s00 * ia + s01 * ic
            gy = s00 * ib + s01 * id_
            gx1 = s10 * ia + s11 * ic
            gy1 = s10 * ib + s11 * id_
            plsc.store_scatter(outb, [rows, _full16(0)], xi0)
            plsc.store_scatter(outb, [rows, _full16(1)], xi1)
            plsc.store_scatter(outb, [rows, _full16(2)], gx)
            plsc.store_scatter(outb, [rows, _full16(3)], gy)
            plsc.store_scatter(outb, [rows, _full16(4)], gx1)
            plsc.store_scatter(outb, [rows, _full16(5)], gy1)
            outp[0, pl.ds(p, 16)] = gx
            outp[1, pl.ds(p, 16)] = gy
            outp[2, pl.ds(p, 16)] = gx1
            outp[3, pl.ds(p, 16)] = gy1
        pltpu.sync_copy(outb, out_t2.at[pl.ds(base, BLK)])
        pltpu.sync_copy(outp, out_gp.at[:, pl.ds(base, BLK)])

    issue(0, sets[0])

    def body(g, carry):
        b0 = 2 * g
        issue(b0 + 1, sets[1])
        wait(sets[0])
        compute(b0, sets[0])

        @pl.when(b0 + 2 < NBLK)
        def _():
            issue(b0 + 2, sets[0])

        wait(sets[1])
        compute(b0 + 1, sets[1])
        return carry

    lax.fori_loop(0, NBLK // 2, body, 0)


def _phase2(table2, gp, nidx, own, out_part,
            idx_a, idx_b, buf_a, buf_b, own_a, own_b, gp_a, gp_b, resv,
            sga, sgb, soa, sob, spa, spb):
    w = _wid()
    iota = lax.iota(jnp.int32, 16)
    sets = [(idx_a, buf_a, own_a, gp_a, sga, soa, spa),
            (idx_b, buf_b, own_b, gp_b, sgb, sob, spb)]

    def issue(b, s):
        idx_v, buf, ownv, gpv, sg, so, sp = s
        pltpu.sync_copy(nidx.at[pl.ds((w * NBLK + b) * IDXB, IDXB)], idx_v)
        pltpu.async_copy(table2.at[idx_v], buf, sg)
        base = w * CPT + b * BLK
        pltpu.async_copy(own.at[:, pl.ds(base, BLK)], ownv, so)
        pltpu.async_copy(gp.at[:, pl.ds(base, BLK)], gpv, sp)

    def wait(s):
        idx_v, buf, ownv, gpv, sg, so, sp = s
        pltpu.make_async_copy(table2.at[idx_v], buf, sg).wait()
        pltpu.make_async_copy(own.at[:, pl.ds(0, BLK)], ownv, so).wait()
        pltpu.make_async_copy(gp.at[:, pl.ds(0, BLK)], gpv, sp).wait()

    def compute(b, s, accf, accu):
        idx_v, buf, ownv, gpv, sg, so, sp = s
        base = w * CPT + b * BLK
        for p in range(0, BLK, 16):
            rows = iota + p
            xi0 = ownv[0, pl.ds(p, 16)]
            xi1 = ownv[1, pl.ds(p, 16)]
            uti = ownv[2, pl.ds(p, 16)]
            u1i = ownv[3, pl.ds(p, 16)]
            gi0 = gpv[0, pl.ds(p, 16)]
            gi1 = gpv[1, pl.ds(p, 16)]
            gi2 = gpv[2, pl.ds(p, 16)]
            gi3 = gpv[3, pl.ds(p, 16)]
            a00 = jnp.zeros((16,), jnp.float32)
            a01 = jnp.zeros((16,), jnp.float32)
            a10 = jnp.zeros((16,), jnp.float32)
            a11 = jnp.zeros((16,), jnp.float32)
            b00 = jnp.zeros((16,), jnp.float32)
            b01 = jnp.zeros((16,), jnp.float32)
            b10 = jnp.zeros((16,), jnp.float32)
            b11 = jnp.zeros((16,), jnp.float32)
            for m in range(M):
                rm = rows + m * BLK
                xj0 = plsc.load_gather(buf, [rm, _full16(0)])
                xj1 = plsc.load_gather(buf, [rm, _full16(1)])
                gj0 = plsc.load_gather(buf, [rm, _full16(2)])
                gj1 = plsc.load_gather(buf, [rm, _full16(3)])
                gj2 = plsc.load_gather(buf, [rm, _full16(4)])
                gj3 = plsc.load_gather(buf, [rm, _full16(5)])
                xd0 = xj0 - xi0
                xd1 = xj1 - xi1
                gd0 = gj0 - gi0
                gd1 = gj1 - gi1
                gd2 = gj2 - gi2
                gd3 = gj3 - gi3
                a00 = a00 + gd0 * xd0
                a01 = a01 + gd0 * xd1
                a10 = a10 + gd1 * xd0
                a11 = a11 + gd1 * xd1
                b00 = b00 + gd2 * xd0
                b01 = b01 + gd2 * xd1
                b10 = b10 + gd3 * xd0
                b11 = b11 + gd3 * xd1
            ia = ownv[4, pl.ds(p, 16)]
            ib = ownv[5, pl.ds(p, 16)]
            ic = ownv[6, pl.ds(p, 16)]
            id_ = ownv[7, pl.ds(p, 16)]
            lap = a00 * ia + a01 * ic + a10 * ib + a11 * id_
            lap1 = b00 * ia + b01 * ic + b10 * ib + b11 * id_
            f = u1i - uti - 0.01 * (
                0.01 * lap + uti - uti * uti * uti
                + 0.01 * lap1 + u1i - u1i * u1i * u1i
            )
            gid = base + p + iota
            f = jnp.where(gid < N, f, 0.0)
            accf = accf + f * f
            upv = ownv[8, pl.ds(p, 16)]
            usv = ownv[9, pl.ds(p, 16)]
            du = upv - usv
            accu = accu + du * du
        return accf, accu

    issue(0, sets[0])
    zero = jnp.zeros((16,), jnp.float32)

    def body(g, carry):
        accf, accu = carry
        b0 = 2 * g
        issue(b0 + 1, sets[1])
        wait(sets[0])
        accf, accu = compute(b0, sets[0], accf, accu)

        @pl.when(b0 + 2 < NBLK)
        def _():
            issue(b0 + 2, sets[0])

        wait(sets[1])
        accf, accu = compute(b0 + 1, sets[1], accf, accu)
        return accf, accu

    accf, accu = lax.fori_loop(0, NBLK // 2, body, (zero, zero))
    resv[...] = accu + 4.0 * accf
    pltpu.sync_copy(resv, out_part.at[w])


def kernel(up, usol, ut, x_to_train_f, ut1, n_index, inv_mat):
    x = x_to_train_f
    pad = NP - N
    table1 = jnp.concatenate([x, ut, ut1], axis=1)
    table1 = jnp.pad(table1, ((0, pad), (0, 0)))
    nidx = jnp.pad(n_index.astype(jnp.int32), ((0, pad), (0, 0)))
    nidx = nidx.reshape(NW * NBLK, BLK, M).transpose(0, 2, 1).reshape(-1)
    inv4 = inv_mat.reshape(N, 4)
    own = jnp.concatenate([x.T, ut.T, ut1.T, inv4.T, up.T, usol.T], axis=0)
    own = jnp.pad(own, ((0, 0), (0, pad)))

    p1 = pl.kernel(
        _phase1,
        out_type=(
            jax.ShapeDtypeStruct((NP, 8), jnp.float32),
            jax.ShapeDtypeStruct((4, NP), jnp.float32),
        ),
        mesh=_make_mesh(),
        compiler_params=_params,
        scratch_types=[
            pltpu.VMEM((IDXB,), jnp.int32),
            pltpu.VMEM((IDXB,), jnp.int32),
            pltpu.VMEM((IDXB, 4), jnp.float32),
            pltpu.VMEM((IDXB, 4), jnp.float32),
            pltpu.VMEM((8, BLK), jnp.float32),
            pltpu.VMEM((8, BLK), jnp.float32),
            pltpu.VMEM((BLK, 8), jnp.float32),
            pltpu.VMEM((4, BLK), jnp.float32),
            pltpu.SemaphoreType.DMA,
            pltpu.SemaphoreType.DMA,
            pltpu.SemaphoreType.DMA,
            pltpu.SemaphoreType.DMA,
        ],
    )
    table2, gp = p1(table1, nidx, own)

    p2 = pl.kernel(
        _phase2,
        out_type=jax.ShapeDtypeStruct((NW, 16), jnp.float32),
        mesh=_make_mesh(),
        compiler_params=_params,
        scratch_types=[
            pltpu.VMEM((IDXB,), jnp.int32),
            pltpu.VMEM((IDXB,), jnp.int32),
            pltpu.VMEM((IDXB, 8), jnp.float32),
            pltpu.VMEM((IDXB, 8), jnp.float32),
            pltpu.VMEM((10, BLK), jnp.float32),
            pltpu.VMEM((10, BLK), jnp.float32),
            pltpu.VMEM((4, BLK), jnp.float32),
            pltpu.VMEM((4, BLK), jnp.float32),
            pltpu.VMEM((16,), jnp.float32),
            pltpu.SemaphoreType.DMA,
            pltpu.SemaphoreType.DMA,
            pltpu.SemaphoreType.DMA,
            pltpu.SemaphoreType.DMA,
            pltpu.SemaphoreType.DMA,
            pltpu.SemaphoreType.DMA,
        ],
    )
    part = p2(table2, gp, nidx, own)
    return jnp.sum(part)
